# baseline (speedup 1.0000x reference)
import jax
import jax.numpy as jnp
from jax.experimental import pallas as pl
from jax.experimental.pallas import tpu as pltpu

NUM_TOKENS = 16384
D_MODEL = 2048
NUM_EXPERTS = 16
TOP_K = 2

BT = 512


def _gate_block(x_ref, w_ref, w_out_ref, idx_out_ref):
    logits = jnp.dot(
        x_ref[...], w_ref[...].T, preferred_element_type=jnp.float32
    )
    lt = logits.T
    iota = jax.lax.broadcasted_iota(jnp.int32, lt.shape, 0)
    l1 = jnp.max(lt, axis=0, keepdims=True)
    i1 = jnp.min(
        jnp.where(lt == l1, iota, NUM_EXPERTS), axis=0, keepdims=True
    )
    masked = jnp.where(iota == i1, -jnp.inf, lt)
    l2 = jnp.max(masked, axis=0, keepdims=True)
    i2 = jnp.min(
        jnp.where(masked == l2, iota, NUM_EXPERTS), axis=0, keepdims=True
    )
    e2 = jnp.exp(l2 - l1)
    s = 1.0 + e2
    w_out_ref[0:1, :] = 1.0 / s
    w_out_ref[1:2, :] = e2 / s
    idx_out_ref[0:1, :] = i1
    idx_out_ref[1:2, :] = i2


def kernel(x, W):
    grid = (NUM_TOKENS // BT,)
    w_pl, idx_pl = pl.pallas_call(
        _gate_block,
        grid=grid,
        in_specs=[
            pl.BlockSpec((BT, D_MODEL), lambda i: (i, 0)),
            pl.BlockSpec((NUM_EXPERTS, D_MODEL), lambda i: (0, 0)),
        ],
        out_specs=[
            pl.BlockSpec((TOP_K, BT), lambda i: (0, i)),
            pl.BlockSpec((TOP_K, BT), lambda i: (0, i)),
        ],
        out_shape=[
            jax.ShapeDtypeStruct((TOP_K, NUM_TOKENS), jnp.float32),
            jax.ShapeDtypeStruct((TOP_K, NUM_TOKENS), jnp.int32),
        ],
        compiler_params=pltpu.CompilerParams(
            dimension_semantics=("parallel",),
        ),
    )(x, W)
    return (w_pl.T, idx_pl.T)


# speedup vs baseline: 1.2018x; 1.2018x over previous
import jax
import jax.numpy as jnp
from jax.experimental import pallas as pl
from jax.experimental.pallas import tpu as pltpu

NUM_TOKENS = 16384
D_MODEL = 2048
NUM_EXPERTS = 16
TOP_K = 2

BT = 1024


def _gate_block(x_ref, w_ref, w_out_ref, idx_out_ref):
    logits = jnp.dot(
        x_ref[...], w_ref[...].T, preferred_element_type=jnp.float32
    )
    lt = logits.T
    iota = jax.lax.broadcasted_iota(jnp.int32, lt.shape, 0)
    l1 = jnp.max(lt, axis=0, keepdims=True)
    i1 = jnp.min(
        jnp.where(lt == l1, iota, NUM_EXPERTS), axis=0, keepdims=True
    )
    masked = jnp.where(iota == i1, -jnp.inf, lt)
    l2 = jnp.max(masked, axis=0, keepdims=True)
    i2 = jnp.min(
        jnp.where(masked == l2, iota, NUM_EXPERTS), axis=0, keepdims=True
    )
    e2 = jnp.exp(l2 - l1)
    s = 1.0 + e2
    w_out_ref[0:1, :] = 1.0 / s
    w_out_ref[1:2, :] = e2 / s
    idx_out_ref[0:1, :] = i1
    idx_out_ref[1:2, :] = i2


def kernel(x, W):
    grid = (NUM_TOKENS // BT,)
    w_pl, idx_pl = pl.pallas_call(
        _gate_block,
        grid=grid,
        in_specs=[
            pl.BlockSpec((BT, D_MODEL), lambda i: (i, 0)),
            pl.BlockSpec((NUM_EXPERTS, D_MODEL), lambda i: (0, 0)),
        ],
        out_specs=[
            pl.BlockSpec((TOP_K, BT), lambda i: (0, i)),
            pl.BlockSpec((TOP_K, BT), lambda i: (0, i)),
        ],
        out_shape=[
            jax.ShapeDtypeStruct((TOP_K, NUM_TOKENS), jnp.float32),
            jax.ShapeDtypeStruct((TOP_K, NUM_TOKENS), jnp.int32),
        ],
        compiler_params=pltpu.CompilerParams(
            dimension_semantics=("arbitrary",),
        ),
    )(x, W)
    return (w_pl.T, idx_pl.T)


# speedup vs baseline: 1.2079x; 1.0050x over previous
import jax
import jax.numpy as jnp
from jax.experimental import pallas as pl
from jax.experimental.pallas import tpu as pltpu

NUM_TOKENS = 16384
D_MODEL = 2048
NUM_EXPERTS = 16
TOP_K = 2

BT = 1024


def _gate_block(x_ref, w_ref, w_out_ref, idx_out_ref):
    logits = jnp.dot(
        x_ref[...], w_ref[...].T, preferred_element_type=jnp.float32
    )
    lt = logits.T
    iota = jax.lax.broadcasted_iota(jnp.int32, lt.shape, 0)
    l1 = jnp.max(lt, axis=0, keepdims=True)
    i1 = jnp.min(
        jnp.where(lt == l1, iota, NUM_EXPERTS), axis=0, keepdims=True
    )
    masked = jnp.where(iota == i1, -jnp.inf, lt)
    l2 = jnp.max(masked, axis=0, keepdims=True)
    i2 = jnp.min(
        jnp.where(masked == l2, iota, NUM_EXPERTS), axis=0, keepdims=True
    )
    e2 = jnp.exp(l2 - l1)
    s = 1.0 + e2
    w_out_ref[0:1, :] = 1.0 / s
    w_out_ref[1:2, :] = e2 / s
    idx_out_ref[0:1, :] = i1
    idx_out_ref[1:2, :] = i2


def kernel(x, W):
    grid = (NUM_TOKENS // BT,)
    w_pl, idx_pl = pl.pallas_call(
        _gate_block,
        grid=grid,
        in_specs=[
            pl.BlockSpec((BT, D_MODEL), lambda i: (i, 0)),
            pl.BlockSpec((NUM_EXPERTS, D_MODEL), lambda i: (0, 0)),
        ],
        out_specs=[
            pl.BlockSpec((TOP_K, BT), lambda i: (0, i)),
            pl.BlockSpec((TOP_K, BT), lambda i: (0, i)),
        ],
        out_shape=[
            jax.ShapeDtypeStruct((TOP_K, NUM_TOKENS), jnp.float32),
            jax.ShapeDtypeStruct((TOP_K, NUM_TOKENS), jnp.int32),
        ],
        compiler_params=pltpu.CompilerParams(
            dimension_semantics=("parallel",),
        ),
    )(x, W)
    return (w_pl.T, idx_pl.T)
